# baseline (device time: 10894 ns/iter reference)
import os

import jax
import jax.numpy as jnp
from jax import lax
from jax.experimental import pallas as pl
from jax.experimental.pallas import tpu as pltpu

Z = 4
ROWS = 512
COLS = 256
B = 160

ABLATE = os.environ.get("ABLATE", "")


def kernel(x, dest):
    dest2 = dest.reshape(1, ROWS).astype(jnp.int32)

    def body(x_ref, d_ref, out_ref, seg_ref, yg_ref, dg_ref,
             sxs, sxr, sds, sdr):
        me_x = lax.axis_index("x")
        me_y = lax.axis_index("y")
        me_z = lax.axis_index("z")

        bar = pltpu.get_barrier_semaphore()
        for dz in range(1, Z):
            pl.semaphore_signal(
                bar, inc=1,
                device_id=(me_x, me_y, lax.rem(me_z + dz, Z)),
                device_id_type=pl.DeviceIdType.MESH,
            )
        pl.semaphore_wait(bar, Z - 1)

        dg_ref[me_z] = d_ref[...]

        d_sends = []
        for k, dz in enumerate(range(1, Z)) if ABLATE != "nocomm" else []:
            p = lax.rem(me_z + dz, Z)
            rd = pltpu.make_async_remote_copy(
                src_ref=dg_ref.at[me_z],
                dst_ref=dg_ref.at[me_z],
                send_sem=sds.at[k],
                recv_sem=sdr.at[k],
                device_id=(me_x, me_y, p),
                device_id_type=pl.DeviceIdType.MESH,
            )
            rd.start()
            d_sends.append(rd)

        d_loc = d_ref[...]
        x_bf = x_ref[...].astype(jnp.bfloat16)
        if ABLATE != "nocompute":
            tri = (
                lax.broadcasted_iota(jnp.int32, (ROWS, ROWS), 0)
                <= lax.broadcasted_iota(jnp.int32, (ROWS, ROWS), 1)
            ).astype(jnp.float32)
            seg_iota = lax.broadcasted_iota(jnp.int32, (B, ROWS), 0)
            for r in range(Z):
                mr = d_loc == r
                cs = lax.dot_general(
                    mr.astype(jnp.float32), tri, (((1,), (0,)), ((), ())),
                    preferred_element_type=jnp.float32,
                )
                key = jnp.where(mr, cs.astype(jnp.int32) - 1, -1)
                p_sel = (seg_iota == key).astype(jnp.bfloat16)
                seg_ref[r] = lax.dot_general(
                    p_sel, x_bf, (((1,), (0,)), ((), ())),
                    preferred_element_type=jnp.float32,
                ).astype(jnp.bfloat16)
        else:
            for r in range(Z):
                seg_ref[r] = x_bf[0:B, :]

        yg_ref[me_z] = seg_ref[me_z]
        x_sends = []
        for k, dz in enumerate(range(1, Z)) if ABLATE != "nocomm" else []:
            p = lax.rem(me_z + dz, Z)
            rx = pltpu.make_async_remote_copy(
                src_ref=seg_ref.at[p],
                dst_ref=yg_ref.at[me_z],
                send_sem=sxs.at[k],
                recv_sem=sxr.at[k],
                device_id=(me_x, me_y, p),
                device_id_type=pl.DeviceIdType.MESH,
            )
            rx.start()
            x_sends.append(rx)

        for dz in range(1, Z) if ABLATE != "nocomm" else []:
            s = lax.rem(me_z + dz, Z)
            ks = Z - dz - 1
            pltpu.make_async_remote_copy(
                src_ref=dg_ref.at[s], dst_ref=dg_ref.at[s],
                send_sem=sds.at[0], recv_sem=sdr.at[ks],
                device_id=(me_x, me_y, s),
                device_id_type=pl.DeviceIdType.MESH,
            ).wait_recv()
            pltpu.make_async_remote_copy(
                src_ref=yg_ref.at[s], dst_ref=yg_ref.at[s],
                send_sem=sxs.at[0], recv_sem=sxr.at[ks],
                device_id=(me_x, me_y, s),
                device_id_type=pl.DeviceIdType.MESH,
            ).wait_recv()

        if ABLATE != "nocompute":
            oi = lax.broadcasted_iota(jnp.int32, (ROWS, B), 0)
            ji = lax.broadcasted_iota(jnp.int32, (ROWS, B), 1)
            acc = jnp.zeros((ROWS, COLS), jnp.float32)
            offset = jnp.int32(0)
            for c in range(Z):
                mc = dg_ref[c] == me_z
                q = (oi == ji + offset).astype(jnp.bfloat16)
                acc = acc + lax.dot_general(
                    q, yg_ref[c], (((1,), (0,)), ((), ())),
                    preferred_element_type=jnp.float32,
                )
                offset = offset + jnp.sum(mc.astype(jnp.int32))
            out_ref[...] = acc
        else:
            for c in range(Z):
                out_ref[pl.ds(c * 128, 128), :] = (
                    yg_ref[c][0:128, :].astype(jnp.float32)
                )

        for rd in d_sends:
            rd.wait_send()
        for rx in x_sends:
            rx.wait_send()

    return pl.pallas_call(
        body,
        out_shape=jax.ShapeDtypeStruct((ROWS, COLS), jnp.float32),
        in_specs=[
            pl.BlockSpec(memory_space=pltpu.VMEM),
            pl.BlockSpec(memory_space=pltpu.VMEM),
        ],
        out_specs=pl.BlockSpec(memory_space=pltpu.VMEM),
        scratch_shapes=[
            pltpu.VMEM((Z, B, COLS), jnp.bfloat16),
            pltpu.VMEM((Z, B, COLS), jnp.bfloat16),
            pltpu.VMEM((Z, 1, ROWS), jnp.int32),
            pltpu.SemaphoreType.DMA((Z - 1,)),
            pltpu.SemaphoreType.DMA((Z - 1,)),
            pltpu.SemaphoreType.DMA((Z - 1,)),
            pltpu.SemaphoreType.DMA((Z - 1,)),
        ],
        compiler_params=pltpu.CompilerParams(collective_id=0),
    )(x, dest2)


# device time: 4204 ns/iter; 2.5913x vs baseline; 2.5913x over previous
import os

import jax
import jax.numpy as jnp
from jax import lax
from jax.experimental import pallas as pl
from jax.experimental.pallas import tpu as pltpu

Z = 4
ROWS = 512
COLS = 256
B = 160

ABLATE = os.environ.get("ABLATE", "")


def kernel(x, dest):
    dest2 = dest.reshape(1, ROWS).astype(jnp.int32)

    def body(x_ref, d_ref, out_ref, seg_ref, yg_ref, dg_ref,
             sxs, sxr, sds, sdr):
        me_x = lax.axis_index("x")
        me_y = lax.axis_index("y")
        me_z = lax.axis_index("z")

        if ABLATE != "pure":
            bar = pltpu.get_barrier_semaphore()
            for dz in range(1, Z):
                pl.semaphore_signal(
                    bar, inc=1,
                    device_id=(me_x, me_y, lax.rem(me_z + dz, Z)),
                    device_id_type=pl.DeviceIdType.MESH,
                )
            pl.semaphore_wait(bar, Z - 1)

        dg_ref[me_z] = d_ref[...]

        d_sends = []
        for k, dz in enumerate(range(1, Z)) if ABLATE not in ("nocomm", "pure") else []:
            p = lax.rem(me_z + dz, Z)
            rd = pltpu.make_async_remote_copy(
                src_ref=dg_ref.at[me_z],
                dst_ref=dg_ref.at[me_z],
                send_sem=sds.at[k],
                recv_sem=sdr.at[k],
                device_id=(me_x, me_y, p),
                device_id_type=pl.DeviceIdType.MESH,
            )
            rd.start()
            d_sends.append(rd)

        d_loc = d_ref[...]
        x_bf = x_ref[...].astype(jnp.bfloat16)
        if ABLATE != "nocompute":
            tri = (
                lax.broadcasted_iota(jnp.int32, (ROWS, ROWS), 0)
                <= lax.broadcasted_iota(jnp.int32, (ROWS, ROWS), 1)
            ).astype(jnp.float32)
            seg_iota = lax.broadcasted_iota(jnp.int32, (B, ROWS), 0)
            for r in range(Z):
                mr = d_loc == r
                cs = lax.dot_general(
                    mr.astype(jnp.float32), tri, (((1,), (0,)), ((), ())),
                    preferred_element_type=jnp.float32,
                )
                key = jnp.where(mr, cs.astype(jnp.int32) - 1, -1)
                p_sel = (seg_iota == key).astype(jnp.bfloat16)
                seg_ref[r] = lax.dot_general(
                    p_sel, x_bf, (((1,), (0,)), ((), ())),
                    preferred_element_type=jnp.float32,
                ).astype(jnp.bfloat16)
        else:
            for r in range(Z):
                seg_ref[r] = x_bf[0:B, :]

        yg_ref[me_z] = seg_ref[me_z]
        x_sends = []
        for k, dz in enumerate(range(1, Z)) if ABLATE not in ("nocomm", "pure") else []:
            p = lax.rem(me_z + dz, Z)
            rx = pltpu.make_async_remote_copy(
                src_ref=seg_ref.at[p],
                dst_ref=yg_ref.at[me_z],
                send_sem=sxs.at[k],
                recv_sem=sxr.at[k],
                device_id=(me_x, me_y, p),
                device_id_type=pl.DeviceIdType.MESH,
            )
            rx.start()
            x_sends.append(rx)

        for dz in range(1, Z) if ABLATE not in ("nocomm", "pure") else []:
            s = lax.rem(me_z + dz, Z)
            ks = Z - dz - 1
            pltpu.make_async_remote_copy(
                src_ref=dg_ref.at[s], dst_ref=dg_ref.at[s],
                send_sem=sds.at[0], recv_sem=sdr.at[ks],
                device_id=(me_x, me_y, s),
                device_id_type=pl.DeviceIdType.MESH,
            ).wait_recv()
            pltpu.make_async_remote_copy(
                src_ref=yg_ref.at[s], dst_ref=yg_ref.at[s],
                send_sem=sxs.at[0], recv_sem=sxr.at[ks],
                device_id=(me_x, me_y, s),
                device_id_type=pl.DeviceIdType.MESH,
            ).wait_recv()

        if ABLATE != "nocompute":
            oi = lax.broadcasted_iota(jnp.int32, (ROWS, B), 0)
            ji = lax.broadcasted_iota(jnp.int32, (ROWS, B), 1)
            acc = jnp.zeros((ROWS, COLS), jnp.float32)
            offset = jnp.int32(0)
            for c in range(Z):
                mc = dg_ref[c] == me_z
                q = (oi == ji + offset).astype(jnp.bfloat16)
                acc = acc + lax.dot_general(
                    q, yg_ref[c], (((1,), (0,)), ((), ())),
                    preferred_element_type=jnp.float32,
                )
                offset = offset + jnp.sum(mc.astype(jnp.int32))
            out_ref[...] = acc
        else:
            for c in range(Z):
                out_ref[pl.ds(c * 128, 128), :] = (
                    yg_ref[c][0:128, :].astype(jnp.float32)
                )

        for rd in d_sends:
            rd.wait_send()
        for rx in x_sends:
            rx.wait_send()

    return pl.pallas_call(
        body,
        out_shape=jax.ShapeDtypeStruct((ROWS, COLS), jnp.float32),
        in_specs=[
            pl.BlockSpec(memory_space=pltpu.VMEM),
            pl.BlockSpec(memory_space=pltpu.VMEM),
        ],
        out_specs=pl.BlockSpec(memory_space=pltpu.VMEM),
        scratch_shapes=[
            pltpu.VMEM((Z, B, COLS), jnp.bfloat16),
            pltpu.VMEM((Z, B, COLS), jnp.bfloat16),
            pltpu.VMEM((Z, 1, ROWS), jnp.int32),
            pltpu.SemaphoreType.DMA((Z - 1,)),
            pltpu.SemaphoreType.DMA((Z - 1,)),
            pltpu.SemaphoreType.DMA((Z - 1,)),
            pltpu.SemaphoreType.DMA((Z - 1,)),
        ],
        compiler_params=(
            pltpu.CompilerParams()
            if ABLATE == "pure"
            else pltpu.CompilerParams(collective_id=0)
        ),
    )(x, dest2)
